# baseline (speedup 1.0000x reference)
import functools

import jax
import jax.numpy as jnp
from jax import lax
from jax.experimental import pallas as pl
from jax.experimental.pallas import tpu as pltpu
from jax.experimental.pallas import tpu_sc as plsc

VOCAB_SIZE = 1000000
EMBED_DIM = 32
NUM_NODES = 16384
OUT_DIM = EMBED_DIM + 2

_NC, _NS, _L = 2, 16, 16
_NW = _NC * _NS
_BPW = NUM_NODES // _NW
_GCHUNK = 128
_NGC = _BPW // _GCHUNK

_mesh = plsc.VectorSubcoreMesh(core_axis_name="c", subcore_axis_name="s")


@functools.partial(
    pl.kernel,
    mesh=_mesh,
    out_type=jax.ShapeDtypeStruct((NUM_NODES * OUT_DIM,), jnp.float32),
    scratch_types=[
        pltpu.VMEM((_BPW,), jnp.int32),
        pltpu.VMEM((_BPW,), jnp.int32),
        pltpu.VMEM((_L,), jnp.float32),
        pltpu.VMEM((_BPW, EMBED_DIM), jnp.float32),
        pltpu.VMEM((_BPW * OUT_DIM,), jnp.float32),
        pltpu.SemaphoreType.DMA,
    ],
    compiler_params=pltpu.CompilerParams(needs_layout_passes=False,
                                         use_tc_tiling_on_sc=False),
)
def _emb_kernel(vocab_hbm, sel_hbm, table_hbm, seltab_hbm, out_hbm,
                idx_v, sid_v, seltab_v, rows_v, stage_v, sem):
    wid = lax.axis_index("s") * _NC + lax.axis_index("c")
    base = wid * _BPW

    pltpu.sync_copy(vocab_hbm.at[pl.ds(base, _BPW)], idx_v)
    pltpu.sync_copy(sel_hbm.at[pl.ds(base, _BPW)], sid_v)
    pltpu.sync_copy(seltab_hbm, seltab_v)

    for j in range(_NGC):
        pltpu.async_copy(table_hbm.at[idx_v.at[pl.ds(j * _GCHUNK, _GCHUNK)]],
                         rows_v.at[pl.ds(j * _GCHUNK, _GCHUNK)], sem)

    tv = seltab_v[...]

    def sel_body(i, _):
        sid = sid_v[pl.ds(i * _L, _L)]
        rvec = lax.iota(jnp.int32, _L) + i * _L
        c0 = tv.at[2 * sid].get(mode="promise_in_bounds")
        c1 = tv.at[2 * sid + 1].get(mode="promise_in_bounds")
        fbase = rvec * OUT_DIM + EMBED_DIM
        plsc.store_scatter(stage_v, [fbase], c0)
        plsc.store_scatter(stage_v, [fbase + 1], c1)
        return 0

    lax.fori_loop(0, _BPW // _L, sel_body, 0)

    for j in range(_NGC):
        pltpu.make_async_copy(table_hbm.at[idx_v.at[pl.ds(j * _GCHUNK, _GCHUNK)]],
                              rows_v.at[pl.ds(j * _GCHUNK, _GCHUNK)], sem).wait()

    def widen_body(r, _):
        stage_v[pl.ds(r * OUT_DIM, _L)] = rows_v[r, pl.ds(0, _L)]
        stage_v[pl.ds(r * OUT_DIM + _L, _L)] = rows_v[r, pl.ds(_L, _L)]
        return 0

    lax.fori_loop(0, _BPW, widen_body, 0)

    pltpu.sync_copy(stage_v, out_hbm.at[pl.ds(base * OUT_DIM, _BPW * OUT_DIM)])


def kernel(vocab_ids, selector_ids, embedding_table, selector_table):
    seltab_flat = jnp.pad(selector_table.reshape(-1).astype(jnp.float32),
                          (0, _L - 4))
    out_flat = _emb_kernel(vocab_ids.astype(jnp.int32),
                           selector_ids.astype(jnp.int32),
                           embedding_table, seltab_flat)
    return out_flat.reshape(NUM_NODES, OUT_DIM)


# speedup vs baseline: 3.8402x; 3.8402x over previous
import functools

import jax
import jax.numpy as jnp
from jax import lax
from jax.experimental import pallas as pl
from jax.experimental.pallas import tpu as pltpu
from jax.experimental.pallas import tpu_sc as plsc

VOCAB_SIZE = 1000000
EMBED_DIM = 32
NUM_NODES = 16384
OUT_DIM = EMBED_DIM + 2

_NC, _NS, _L = 2, 16, 16
_NW = _NC * _NS
_BPW = NUM_NODES // _NW
_CL = 8
_NCHUNK = _BPW // _CL
_CB = EMBED_DIM // 8
_TW = 128
_NBUFC = 2
_RROWS = _CL * EMBED_DIM

_mesh = plsc.VectorSubcoreMesh(core_axis_name="c", subcore_axis_name="s")


@functools.partial(
    pl.kernel,
    mesh=_mesh,
    out_type=jax.ShapeDtypeStruct((OUT_DIM, NUM_NODES), jnp.float32),
    scratch_types=[
        pltpu.VMEM((_BPW + _L,), jnp.int32),
        pltpu.VMEM((_BPW,), jnp.int32),
        pltpu.VMEM((_L,), jnp.float32),
        pltpu.VMEM((_NBUFC * _RROWS, _TW), jnp.float32),
        pltpu.VMEM((_STG := 40, _BPW), jnp.float32),
        pltpu.SemaphoreType.DMA,
        pltpu.SemaphoreType.DMA,
    ],
    compiler_params=pltpu.CompilerParams(needs_layout_passes=False,
                                         use_tc_tiling_on_sc=True),
)
def _emb_kernel(vocab_hbm, sel_hbm, table3_hbm, seltab_hbm, out_t_hbm,
                idx_v, sid_v, seltab_v, ring_v, stage_v, sem0, sem1):
    wid = lax.axis_index("s") * _NC + lax.axis_index("c")
    base = wid * _BPW

    pltpu.sync_copy(vocab_hbm.at[pl.ds(base, _BPW)], idx_v.at[pl.ds(0, _BPW)])
    pltpu.sync_copy(sel_hbm.at[pl.ds(base, _BPW)], sid_v)
    pltpu.sync_copy(seltab_hbm, seltab_v)

    lanes = lax.iota(jnp.int32, _L)

    def fire_chunk(k, slot, sem):
        iv = idx_v[pl.ds(k * _CL, _L)]
        for sub in range(_CL):
            r = iv[sub]
            q = pl.multiple_of((r // _TW) * _TW, _TW)
            for cb in range(_CB):
                pltpu.async_copy(
                    table3_hbm.at[cb, pl.ds(0, 8), pl.ds(q, _TW)],
                    ring_v.at[pl.ds(slot * _RROWS + sub * EMBED_DIM + cb * 8,
                                    8), pl.ds(0, _TW)],
                    sem)

    def drain_chunk(sem):
        for _ in range(_CL * _CB):
            pltpu.make_async_copy(
                table3_hbm.at[0, pl.ds(0, 8), pl.ds(0, _TW)],
                ring_v.at[pl.ds(0, 8), pl.ds(0, _TW)], sem).wait()

    def extract_chunk(k, slot):
        iv = idx_v[pl.ds(k * _CL, _L)]
        sub = lax.rem(lanes, jnp.full((_L,), _CL, jnp.int32))
        mfull = lax.rem(iv, jnp.full((_L,), _TW, jnp.int32))
        mvec = mfull.at[sub].get(mode="promise_in_bounds")
        half = lanes // _CL
        for c in range(0, EMBED_DIM, 2):
            rows = slot * _RROWS + sub * EMBED_DIM + c + half
            vals = plsc.load_gather(ring_v, [rows, mvec])
            plsc.store_scatter(stage_v, [c + half, k * _CL + sub], vals)

    fire_chunk(0, 0, sem0)
    fire_chunk(1, 1, sem1)

    def loop_body(k2, _):
        k = 2 * k2
        drain_chunk(sem0)
        extract_chunk(k, 0)
        fire_chunk(k + 2, 0, sem0)
        drain_chunk(sem1)
        extract_chunk(k + 1, 1)
        fire_chunk(k + 3, 1, sem1)
        return 0

    lax.fori_loop(0, _NCHUNK // 2 - 1, loop_body, 0)

    drain_chunk(sem0)
    extract_chunk(_NCHUNK - 2, 0)
    drain_chunk(sem1)
    extract_chunk(_NCHUNK - 1, 1)

    tv = seltab_v[...]

    def sel_body(i, _):
        sid = sid_v[pl.ds(i * _L, _L)]
        stage_v[EMBED_DIM, pl.ds(i * _L, _L)] = tv.at[2 * sid].get(
            mode="promise_in_bounds")
        stage_v[EMBED_DIM + 1, pl.ds(i * _L, _L)] = tv.at[2 * sid + 1].get(
            mode="promise_in_bounds")
        return 0

    lax.fori_loop(0, _BPW // _L, sel_body, 0)

    for g in range(_CB):
        pltpu.sync_copy(
            stage_v.at[pl.ds(g * 8, 8), pl.ds(0, _BPW)],
            out_t_hbm.at[pl.ds(g * 8, 8), pl.ds(base, _BPW)])
    pltpu.sync_copy(stage_v.at[pl.ds(EMBED_DIM, 2), pl.ds(0, _BPW)],
                    out_t_hbm.at[pl.ds(EMBED_DIM, 2), pl.ds(base, _BPW)])


def kernel(vocab_ids, selector_ids, embedding_table, selector_table):
    seltab_flat = jnp.pad(selector_table.reshape(-1).astype(jnp.float32),
                          (0, _L - 4))
    table3 = embedding_table.T.reshape(_CB, 8, VOCAB_SIZE)
    out_t = _emb_kernel(vocab_ids.astype(jnp.int32),
                        selector_ids.astype(jnp.int32),
                        table3, seltab_flat)
    return out_t.T


# speedup vs baseline: 3.9073x; 1.0175x over previous
import functools

import jax
import jax.numpy as jnp
from jax import lax
from jax.experimental import pallas as pl
from jax.experimental.pallas import tpu as pltpu
from jax.experimental.pallas import tpu_sc as plsc

VOCAB_SIZE = 1000000
EMBED_DIM = 32
NUM_NODES = 16384
OUT_DIM = EMBED_DIM + 2

_NC, _NS, _L = 2, 16, 16
_NW = _NC * _NS
_BPW = NUM_NODES // _NW
_CL = 8
_NCHUNK = _BPW // _CL
_CB = EMBED_DIM // 8
_TW = 128
_NBUFC = 2
_RROWS = _CL * EMBED_DIM

_mesh = plsc.VectorSubcoreMesh(core_axis_name="c", subcore_axis_name="s")


@functools.partial(
    pl.kernel,
    mesh=_mesh,
    out_type=jax.ShapeDtypeStruct((OUT_DIM, NUM_NODES), jnp.float32),
    scratch_types=[
        pltpu.VMEM((_BPW + _L,), jnp.int32),
        pltpu.VMEM((_BPW,), jnp.int32),
        pltpu.VMEM((_L,), jnp.float32),
        pltpu.VMEM((_NBUFC * _CL, _CB, 8, _TW), jnp.float32),
        pltpu.VMEM((_STG := 40, _BPW), jnp.float32),
        pltpu.SemaphoreType.DMA,
        pltpu.SemaphoreType.DMA,
    ],
    compiler_params=pltpu.CompilerParams(needs_layout_passes=False,
                                         use_tc_tiling_on_sc=True),
)
def _emb_kernel(vocab_hbm, sel_hbm, table3_hbm, seltab_hbm, out_t_hbm,
                idx_v, sid_v, seltab_v, ring_v, stage_v, sem0, sem1):
    wid = lax.axis_index("s") * _NC + lax.axis_index("c")
    base = wid * _BPW

    pltpu.sync_copy(vocab_hbm.at[pl.ds(base, _BPW)], idx_v.at[pl.ds(0, _BPW)])
    pltpu.sync_copy(sel_hbm.at[pl.ds(base, _BPW)], sid_v)
    pltpu.sync_copy(seltab_hbm, seltab_v)

    lanes = lax.iota(jnp.int32, _L)

    def fire_chunk(k, slot, sem):
        iv = idx_v[pl.ds(k * _CL, _L)]
        for sub in range(_CL):
            r = iv[sub]
            q = pl.multiple_of((r // _TW) * _TW, _TW)
            pltpu.async_copy(
                table3_hbm.at[pl.ds(0, _CB), pl.ds(0, 8), pl.ds(q, _TW)],
                ring_v.at[slot * _CL + sub], sem)

    def drain_chunk(sem):
        for _ in range(_CL):
            pltpu.make_async_copy(
                table3_hbm.at[pl.ds(0, _CB), pl.ds(0, 8), pl.ds(0, _TW)],
                ring_v.at[0], sem).wait()

    def extract_chunk(k, slot):
        iv = idx_v[pl.ds(k * _CL, _L)]
        sub = lax.rem(lanes, jnp.full((_L,), _CL, jnp.int32))
        mfull = lax.rem(iv, jnp.full((_L,), _TW, jnp.int32))
        mvec = mfull.at[sub].get(mode="promise_in_bounds")
        half = lanes // _CL
        for c in range(0, EMBED_DIM, 2):
            d = c + half
            vals = plsc.load_gather(
                ring_v, [slot * _CL + sub, d // 8, lax.rem(d, jnp.full(
                    (_L,), 8, jnp.int32)), mvec])
            plsc.store_scatter(stage_v, [d, k * _CL + sub], vals)

    fire_chunk(0, 0, sem0)
    fire_chunk(1, 1, sem1)

    def loop_body(k2, _):
        k = 2 * k2
        drain_chunk(sem0)
        extract_chunk(k, 0)
        fire_chunk(k + 2, 0, sem0)
        drain_chunk(sem1)
        extract_chunk(k + 1, 1)
        fire_chunk(k + 3, 1, sem1)
        return 0

    lax.fori_loop(0, _NCHUNK // 2 - 1, loop_body, 0)

    drain_chunk(sem0)
    extract_chunk(_NCHUNK - 2, 0)
    drain_chunk(sem1)
    extract_chunk(_NCHUNK - 1, 1)

    tv = seltab_v[...]

    def sel_body(i, _):
        sid = sid_v[pl.ds(i * _L, _L)]
        stage_v[EMBED_DIM, pl.ds(i * _L, _L)] = tv.at[2 * sid].get(
            mode="promise_in_bounds")
        stage_v[EMBED_DIM + 1, pl.ds(i * _L, _L)] = tv.at[2 * sid + 1].get(
            mode="promise_in_bounds")
        return 0

    lax.fori_loop(0, _BPW // _L, sel_body, 0)

    for g in range(_CB):
        pltpu.sync_copy(
            stage_v.at[pl.ds(g * 8, 8), pl.ds(0, _BPW)],
            out_t_hbm.at[pl.ds(g * 8, 8), pl.ds(base, _BPW)])
    pltpu.sync_copy(stage_v.at[pl.ds(EMBED_DIM, 2), pl.ds(0, _BPW)],
                    out_t_hbm.at[pl.ds(EMBED_DIM, 2), pl.ds(base, _BPW)])


def kernel(vocab_ids, selector_ids, embedding_table, selector_table):
    seltab_flat = jnp.pad(selector_table.reshape(-1).astype(jnp.float32),
                          (0, _L - 4))
    table3 = embedding_table.T.reshape(_CB, 8, VOCAB_SIZE)
    out_t = _emb_kernel(vocab_ids.astype(jnp.int32),
                        selector_ids.astype(jnp.int32),
                        table3, seltab_flat)
    return out_t.T
